# baseline (speedup 1.0000x reference)
import functools

import jax
import jax.numpy as jnp
from jax import lax
from jax.experimental import pallas as pl
from jax.experimental.pallas import tpu as pltpu
from jax.experimental.pallas import tpu_sc as plsc

_VOCAB = 100000
_D = 300
_DP = 384
_B = 1024
_CTX = 20

_L = 16
_NFULL = _D // _L
_TAIL = _D - _L
_NW = 32
_BPW = _B // _NW
_GCHUNK = 4
_NCHUNK = _BPW // _GCHUNK

_CPR = 80
_NCOPY = _VOCAB // _CPR


def _lane_sum(x):
    lanes = lax.iota(jnp.int32, _L)
    dnums = lax.GatherDimensionNumbers(
        offset_dims=(), collapsed_slice_dims=(0,), start_index_map=(0,))
    for k in (8, 4, 2, 1):
        perm = (lanes ^ k).reshape(_L, 1)
        x = x + lax.gather(x, perm, dnums, slice_sizes=(1,),
                           mode=lax.GatherScatterMode.PROMISE_IN_BOUNDS)
    return x


def _nr_rsqrt(x):
    i = lax.bitcast_convert_type(x, jnp.int32)
    y = lax.bitcast_convert_type(
        jnp.int32(0x5F3759DF) - lax.shift_right_logical(i, 1), jnp.float32)
    for _ in range(4):
        y = y * (1.5 - 0.5 * x * y * y)
    return y


def _sc_pad_copy(emb_table):
    mesh = plsc.VectorSubcoreMesh(core_axis_name="c", subcore_axis_name="s")

    @functools.partial(
        pl.kernel,
        mesh=mesh,
        out_type=jax.ShapeDtypeStruct((_VOCAB, _DP), jnp.float32),
        scratch_types=[
            pltpu.VMEM((_CPR, _D), jnp.float32),
            pltpu.VMEM((_CPR, _DP), jnp.float32),
            pltpu.SemaphoreType.DMA,
        ],
    )
    def body(table_hbm, out_hbm, in_v, out_v, sem):
        wid = lax.axis_index("s") * 2 + lax.axis_index("c")
        nloop = (_NCOPY + _NW - 1) // _NW

        def chunk_body(k, carry):
            cid = k * _NW + wid

            @pl.when(cid < _NCOPY)
            def _():
                r0 = cid * _CPR
                pltpu.sync_copy(table_hbm.at[pl.ds(r0, _CPR)], in_v)

                def row_body(r, carry2):
                    for j in range(_NFULL):
                        out_v[r, pl.ds(j * _L, _L)] = in_v[r, pl.ds(j * _L, _L)]
                    out_v[r, pl.ds(_TAIL, _L)] = in_v[r, pl.ds(_TAIL, _L)]
                    return carry2

                lax.fori_loop(0, _CPR, row_body, 0)
                pltpu.sync_copy(out_v, out_hbm.at[pl.ds(r0, _CPR)])
            return carry

        lax.fori_loop(0, nloop, chunk_body, 0)

    return body(emb_table)


def _sc_pool(idx_flat, table_pad):
    mesh = plsc.VectorSubcoreMesh(core_axis_name="c", subcore_axis_name="s")

    @functools.partial(
        pl.kernel,
        mesh=mesh,
        out_type=jax.ShapeDtypeStruct((_B, _DP), jnp.float32),
        scratch_types=[
            pltpu.VMEM((_BPW * _CTX,), jnp.int32),
            pltpu.VMEM((_GCHUNK * _CTX, _DP), jnp.float32),
            pltpu.VMEM((_BPW, _DP), jnp.float32),
            pltpu.SemaphoreType.DMA,
        ],
    )
    def body(idx_hbm, table_hbm, out_hbm, idx_v, rows_v, outb_v, sem):
        wid = lax.axis_index("s") * 2 + lax.axis_index("c")
        nidx = _BPW * _CTX
        nacc = _NFULL + 1
        last_mask = lax.iota(jnp.int32, _L) < (_D - _NFULL * _L)

        pltpu.sync_copy(idx_hbm.at[pl.ds(wid * nidx, nidx)], idx_v)

        def chunk_body(c, carry):
            pltpu.async_copy(
                table_hbm.at[idx_v.at[pl.ds(c * (_GCHUNK * _CTX),
                                            _GCHUNK * _CTX)]],
                rows_v, sem).wait()

            def batch_body(b, carry2):
                row0 = b * _CTX

                def row_body(r, accs):
                    row = row0 + r
                    vs = [rows_v[row, pl.ds(j * _L, _L)]
                          for j in range(nacc)]
                    ssv = vs[0] * vs[0]
                    for j in range(1, _NFULL):
                        ssv = ssv + vs[j] * vs[j]
                    tm = jnp.where(last_mask, vs[_NFULL], 0.0)
                    ssv = ssv + tm * tm
                    scale = jnp.minimum(_nr_rsqrt(_lane_sum(ssv)), 1.0)
                    return tuple(accs[j] + scale * vs[j]
                                 for j in range(nacc))

                accs0 = tuple(jnp.zeros((_L,), jnp.float32)
                              for _ in range(nacc))
                accs = lax.fori_loop(0, _CTX, row_body, accs0)
                gb = c * _GCHUNK + b
                inv = jnp.float32(1.0 / _CTX)
                for j in range(nacc):
                    outb_v[gb, pl.ds(j * _L, _L)] = accs[j] * inv
                return carry2

            lax.fori_loop(0, _GCHUNK, batch_body, 0)
            return carry

        lax.fori_loop(0, _NCHUNK, chunk_body, 0)
        pltpu.sync_copy(outb_v, out_hbm.at[pl.ds(wid * _BPW, _BPW)])

    return body(idx_flat, table_pad)


_VB = 2048
_NVB = pl.cdiv(_VOCAB, _VB)


def _proj_body(h_ref, w_ref, b_ref, o_ref):
    o_ref[...] = lax.dot_general(
        h_ref[:, : _D], w_ref[...],
        dimension_numbers=(((1,), (1,)), ((), ())),
        preferred_element_type=jnp.float32,
    ) + b_ref[...]


def _projection(hidden, lin_w, lin_b2d):
    return pl.pallas_call(
        _proj_body,
        grid=(_NVB,),
        in_specs=[
            pl.BlockSpec((_B, _DP), lambda i: (0, 0)),
            pl.BlockSpec((_VB, _D), lambda i: (i, 0)),
            pl.BlockSpec((1, _VB), lambda i: (0, i)),
        ],
        out_specs=pl.BlockSpec((_B, _VB), lambda i: (0, i)),
        out_shape=jax.ShapeDtypeStruct((_B, _VOCAB), jnp.float32),
        compiler_params=pltpu.CompilerParams(
            dimension_semantics=("arbitrary",)),
    )(hidden, lin_w, lin_b2d)


def kernel(inputs_, emb_table, lin_w, lin_b):
    idx_flat = inputs_.astype(jnp.int32).reshape(-1)
    table_pad = _sc_pad_copy(emb_table)
    hidden = _sc_pool(idx_flat, table_pad)
    return _projection(hidden, lin_w, lin_b.reshape(1, _VOCAB))


# speedup vs baseline: 1.0799x; 1.0799x over previous
import functools

import jax
import jax.numpy as jnp
from jax import lax
from jax.experimental import pallas as pl
from jax.experimental.pallas import tpu as pltpu
from jax.experimental.pallas import tpu_sc as plsc

_VOCAB = 100000
_D = 300
_DP = 384
_B = 1024
_CTX = 20

_L = 16
_NV = _DP // _L
_NW = 32
_BPW = _B // _NW
_GCHUNK = 4
_NCHUNK = _BPW // _GCHUNK

_RB = 2000


def _renorm_body(t_ref, o_ref):
    x = t_ref[...]
    ss = jnp.sum(x * x, axis=1, keepdims=True)
    scale = jnp.where(ss > 1.0, lax.rsqrt(ss), 1.0)
    o_ref[...] = jnp.concatenate(
        [x * scale, jnp.zeros((_RB, _DP - _D), jnp.float32)], axis=1)


def _renorm_pad(emb_table):
    return pl.pallas_call(
        _renorm_body,
        grid=(_VOCAB // _RB,),
        in_specs=[pl.BlockSpec((_RB, _D), lambda i: (i, 0))],
        out_specs=pl.BlockSpec((_RB, _DP), lambda i: (i, 0)),
        out_shape=jax.ShapeDtypeStruct((_VOCAB, _DP), jnp.float32),
        compiler_params=pltpu.CompilerParams(
            dimension_semantics=("arbitrary",)),
    )(emb_table)


def _sc_pool(idx_flat, table_pad):
    mesh = plsc.VectorSubcoreMesh(core_axis_name="c", subcore_axis_name="s")

    @functools.partial(
        pl.kernel,
        mesh=mesh,
        out_type=jax.ShapeDtypeStruct((_B, _DP), jnp.float32),
        scratch_types=[
            pltpu.VMEM((_BPW * _CTX,), jnp.int32),
            pltpu.VMEM((_GCHUNK * _CTX, _DP), jnp.float32),
            pltpu.VMEM((_BPW, _DP), jnp.float32),
            pltpu.SemaphoreType.DMA,
        ],
    )
    def body(idx_hbm, table_hbm, out_hbm, idx_v, rows_v, outb_v, sem):
        wid = lax.axis_index("s") * 2 + lax.axis_index("c")
        pltpu.sync_copy(idx_hbm.at[pl.ds(wid * (_BPW * _CTX), _BPW * _CTX)],
                        idx_v)

        def chunk_body(c, carry):
            pltpu.async_copy(
                table_hbm.at[idx_v.at[pl.ds(c * (_GCHUNK * _CTX),
                                            _GCHUNK * _CTX)]],
                rows_v, sem).wait()

            def batch_body(b, carry2):
                row0 = b * _CTX

                def row_body(r, accs):
                    row = row0 + r
                    return tuple(
                        accs[j] + rows_v[row, pl.ds(j * _L, _L)]
                        for j in range(_NV))

                accs0 = tuple(jnp.zeros((_L,), jnp.float32)
                              for _ in range(_NV))
                accs = lax.fori_loop(0, _CTX, row_body, accs0)
                gb = c * _GCHUNK + b
                inv = jnp.float32(1.0 / _CTX)
                for j in range(_NV):
                    outb_v[gb, pl.ds(j * _L, _L)] = accs[j] * inv
                return carry2

            lax.fori_loop(0, _GCHUNK, batch_body, 0)
            return carry

        lax.fori_loop(0, _NCHUNK, chunk_body, 0)
        pltpu.sync_copy(outb_v, out_hbm.at[pl.ds(wid * _BPW, _BPW)])

    return body(idx_flat, table_pad)


_VB = 4096
_NVB = pl.cdiv(_VOCAB, _VB)


def _proj_body(h_ref, w_ref, b_ref, o_ref):
    h = h_ref[:, : _D].astype(jnp.bfloat16)
    w = w_ref[...].astype(jnp.bfloat16)
    o_ref[...] = lax.dot_general(
        h, w,
        dimension_numbers=(((1,), (1,)), ((), ())),
        preferred_element_type=jnp.float32,
    ) + b_ref[...]


def _projection(hidden, lin_w, lin_b2d):
    return pl.pallas_call(
        _proj_body,
        grid=(_NVB,),
        in_specs=[
            pl.BlockSpec((_B, _DP), lambda i: (0, 0)),
            pl.BlockSpec((_VB, _D), lambda i: (i, 0)),
            pl.BlockSpec((1, _VB), lambda i: (0, i)),
        ],
        out_specs=pl.BlockSpec((_B, _VB), lambda i: (0, i)),
        out_shape=jax.ShapeDtypeStruct((_B, _VOCAB), jnp.float32),
        compiler_params=pltpu.CompilerParams(
            dimension_semantics=("arbitrary",)),
    )(hidden, lin_w, lin_b2d)


def kernel(inputs_, emb_table, lin_w, lin_b):
    idx_flat = inputs_.astype(jnp.int32).reshape(-1)
    table_pad = _renorm_pad(emb_table)
    hidden = _sc_pool(idx_flat, table_pad)
    return _projection(hidden, lin_w, lin_b.reshape(1, _VOCAB))
